# baseline (speedup 1.0000x reference)
import jax
import jax.numpy as jnp
from jax.experimental import pallas as pl
from jax.experimental.pallas import tpu as pltpu

_TM = 2048
_NS = 4


def _embed_kernel(a0, a1, a2, a3, we_ref, out_ref):
    w = we_ref[...].astype(jnp.bfloat16)
    h = a0.shape[0]
    for idx, a in enumerate((a0, a1, a2, a3)):
        out_ref[idx * h:(idx + 1) * h, :] = jnp.dot(
            a[...].astype(jnp.bfloat16), w, preferred_element_type=jnp.float32
        )


def kernel(adj, W_E):
    B, N, N2 = adj.shape
    D = W_E.shape[1]
    M = B * N
    adj2 = adj.reshape(M, N)
    th = _TM // _NS

    def mk(idx):
        return pl.BlockSpec((th, N), lambda i, idx=idx: (_NS * i + idx, 0))

    out = pl.pallas_call(
        _embed_kernel,
        out_shape=jax.ShapeDtypeStruct((M, D), jnp.float32),
        grid=(M // _TM,),
        in_specs=[mk(0), mk(1), mk(2), mk(3),
                  pl.BlockSpec((N, D), lambda i: (0, 0))],
        out_specs=pl.BlockSpec((_TM, D), lambda i: (i, 0)),
        compiler_params=pltpu.CompilerParams(
            dimension_semantics=("parallel",),
        ),
        cost_estimate=pl.CostEstimate(
            flops=2 * M * N * D,
            transcendentals=0,
            bytes_accessed=adj.size * 4 + W_E.size * 4 + M * D * 4,
        ),
    )(adj2, adj2, adj2, adj2, W_E)

    return out.reshape(B, N, D)
